# baseline (speedup 1.0000x reference)
import dataclasses
import functools

import jax
import jax.numpy as jnp
from jax import lax
from jax.experimental import pallas as pl
from jax.experimental.pallas import tpu as pltpu
from jax.experimental.pallas import tpu_sc as plsc

_K = 11
_BN = 2048
_NW = 32


def _topk_body(p_ref, st_ref, idx_ref):
    p = p_ref[0]
    st = st_ref[0]
    m = st.shape[1]
    bn = p.shape[0]
    nc = m // 128
    dist = None
    for d in range(3):
        diff = p[:, d:d + 1] - st[d:d + 1, :]
        dist = diff * diff if dist is None else dist + diff * diff
    bits = lax.bitcast_convert_type(dist, jnp.int32)
    chunks = [
        (bits[:, c * 128:(c + 1) * 128] & jnp.int32(~15)) | jnp.int32(c)
        for c in range(nc)
    ]
    lane_iota = lax.broadcasted_iota(jnp.int32, (bn, 128), 1)
    imax = jnp.int32(2147483647)
    prev = None
    for k in range(_K):
        acc = None
        for c in range(nc):
            cand = (chunks[c] if prev is None
                    else jnp.where(chunks[c] > prev, chunks[c], imax))
            acc = cand if acc is None else jnp.minimum(acc, cand)
        mn = jnp.min(acc, axis=1, keepdims=True)
        lane = jnp.min(jnp.where(acc == mn, lane_iota, 128),
                       axis=1, keepdims=True)
        idx_ref[0, :, k:k + 1] = (mn & 15) * 128 + lane
        prev = mn


def _topk_indices(points, spoints_t, interpret=False):
    b, n, _ = points.shape
    m = spoints_t.shape[2]
    return pl.pallas_call(
        _topk_body,
        grid=(b, n // _BN),
        in_specs=[
            pl.BlockSpec((1, _BN, 3), lambda bi, i: (bi, i, 0)),
            pl.BlockSpec((1, 3, m), lambda bi, i: (bi, 0, 0)),
        ],
        out_specs=pl.BlockSpec((1, _BN, _K), lambda bi, i: (bi, i, 0)),
        out_shape=jax.ShapeDtypeStruct((b, n, _K), jnp.int32),
        interpret=interpret,
    )(points, spoints_t)


def _edge_min(idx_w, points_t, spoints_t):
    b, _, n = points_t.shape
    m = spoints_t.shape[2]
    total = b * n
    ppw = total // _NW
    wpb = _NW // b
    mesh = plsc.VectorSubcoreMesh(core_axis_name="c", subcore_axis_name="s")
    cp = pltpu.CompilerParams()
    if "needs_layout_passes" in pltpu.CompilerParams.__dataclass_fields__:
        cp = dataclasses.replace(cp, needs_layout_passes=False)

    @functools.partial(
        pl.kernel,
        out_type=jax.ShapeDtypeStruct((total,), jnp.float32),
        mesh=mesh,
        compiler_params=cp,
        scratch_types=[
            pltpu.VMEM((3, m), jnp.float32),
            pltpu.VMEM((3, ppw), jnp.float32),
            pltpu.VMEM((_K, ppw), jnp.int32),
            pltpu.VMEM((ppw,), jnp.float32),
        ],
    )
    def body(idx_hbm, p_hbm, s_hbm, out_hbm, sv, pv, iv, ov):
        wid = lax.axis_index("s") * 2 + lax.axis_index("c")
        bi = wid // wpb
        nbase = (wid % wpb) * ppw
        base = wid * ppw
        pltpu.sync_copy(s_hbm.at[bi], sv)
        pltpu.sync_copy(p_hbm.at[bi, :, pl.ds(nbase, ppw)], pv)
        pltpu.sync_copy(idx_hbm.at[wid], iv)

        row0 = jnp.zeros((16,), jnp.int32)
        row1 = jnp.full((16,), 1, jnp.int32)
        row2 = jnp.full((16,), 2, jnp.int32)

        @pl.loop(0, ppw, step=16)
        def _chunk(c):
            sl = pl.ds(c, 16)
            pxv, pyv, pzv = pv[0, sl], pv[1, sl], pv[2, sl]
            i0 = iv[0, sl]
            c0x = plsc.load_gather(sv, [row0, i0])
            c0y = plsc.load_gather(sv, [row1, i0])
            c0z = plsc.load_gather(sv, [row2, i0])
            d0x, d0y, d0z = pxv - c0x, pyv - c0y, pzv - c0z
            d0 = d0x * d0x + d0y * d0y + d0z * d0z
            acc = jnp.full((16,), jnp.inf, jnp.float32)
            for j in range(1, _K):
                ij = iv[j, sl]
                cjx = plsc.load_gather(sv, [row0, ij])
                cjy = plsc.load_gather(sv, [row1, ij])
                cjz = plsc.load_gather(sv, [row2, ij])
                ex, ey, ez = cjx - c0x, cjy - c0y, cjz - c0z
                lsq = ex * ex + ey * ey + ez * ez
                qx, qy, qz = pxv - cjx, pyv - cjy, pzv - cjz
                dj = qx * qx + qy * qy + qz * qz
                diff = dj - d0
                val = (diff * diff) / (4.0 * lsq)
                acc = jnp.minimum(acc, val)
            ov[sl] = acc

        pltpu.sync_copy(ov, out_hbm.at[pl.ds(base, ppw)])

    return body(idx_w, points_t, spoints_t)


def kernel(points, spoints):
    b, n, _ = points.shape
    total = b * n
    ppw = total // _NW
    spoints_t = jnp.transpose(spoints, (0, 2, 1))
    idx = _topk_indices(points, spoints_t)
    idx_w = idx.reshape(_NW, ppw, _K).transpose(0, 2, 1)
    points_t = jnp.transpose(points, (0, 2, 1))
    out = _edge_min(idx_w, points_t, spoints_t)
    return out.reshape(b, n)


# speedup vs baseline: 1.7626x; 1.7626x over previous
import dataclasses
import functools

import jax
import jax.numpy as jnp
from jax import lax
from jax.experimental import pallas as pl
from jax.experimental.pallas import tpu as pltpu
from jax.experimental.pallas import tpu_sc as plsc

_K = 11
_BN = 1024
_NW = 32


def _topk_body(p_ref, st_ref, idx_ref):
    p = p_ref[0]
    st = st_ref[0]
    m = st.shape[1]
    bn = p.shape[0]
    nc = m // 128
    dist = None
    for d in range(3):
        diff = p[:, d:d + 1] - st[d:d + 1, :]
        dist = diff * diff if dist is None else dist + diff * diff
    bits = lax.bitcast_convert_type(dist, jnp.int32)
    chunks = [
        lax.bitcast_convert_type(
            (bits[:, c * 128:(c + 1) * 128] & jnp.int32(~15)) | jnp.int32(c),
            jnp.float32)
        for c in range(nc)
    ]
    lane_iota = lax.convert_element_type(
        lax.broadcasted_iota(jnp.int32, (bn, 128), 1), jnp.float32)
    inf = jnp.float32(jnp.inf)
    prev = None
    for k in range(_K):
        acc = None
        for c in range(nc):
            cand = (chunks[c] if prev is None
                    else jnp.where(chunks[c] > prev, chunks[c], inf))
            acc = cand if acc is None else jnp.minimum(acc, cand)
        mn = jnp.min(acc, axis=1, keepdims=True)
        lane = lax.convert_element_type(
            jnp.min(jnp.where(acc == mn, lane_iota, jnp.float32(128.0)),
                    axis=1, keepdims=True), jnp.int32)
        mn_bits = lax.bitcast_convert_type(mn, jnp.int32)
        idx_ref[0, :, k:k + 1] = (mn_bits & 15) * 128 + lane
        prev = mn


def _topk_indices(points, spoints_t, interpret=False):
    b, n, _ = points.shape
    m = spoints_t.shape[2]
    return pl.pallas_call(
        _topk_body,
        grid=(b, n // _BN),
        in_specs=[
            pl.BlockSpec((1, _BN, 3), lambda bi, i: (bi, i, 0)),
            pl.BlockSpec((1, 3, m), lambda bi, i: (bi, 0, 0)),
        ],
        out_specs=pl.BlockSpec((1, _BN, _K), lambda bi, i: (bi, i, 0)),
        out_shape=jax.ShapeDtypeStruct((b, n, _K), jnp.int32),
        interpret=interpret,
    )(points, spoints_t)


def _edge_min(idx_w, points_t, spoints_t):
    b, _, n = points_t.shape
    m = spoints_t.shape[2]
    total = b * n
    ppw = total // _NW
    wpb = _NW // b
    mesh = plsc.VectorSubcoreMesh(core_axis_name="c", subcore_axis_name="s")
    cp = pltpu.CompilerParams()
    if "needs_layout_passes" in pltpu.CompilerParams.__dataclass_fields__:
        cp = dataclasses.replace(cp, needs_layout_passes=False)

    @functools.partial(
        pl.kernel,
        out_type=jax.ShapeDtypeStruct((total,), jnp.float32),
        mesh=mesh,
        compiler_params=cp,
        scratch_types=[
            pltpu.VMEM((3, m), jnp.float32),
            pltpu.VMEM((3, ppw), jnp.float32),
            pltpu.VMEM((_K, ppw), jnp.int32),
            pltpu.VMEM((ppw,), jnp.float32),
        ],
    )
    def body(idx_hbm, p_hbm, s_hbm, out_hbm, sv, pv, iv, ov):
        wid = lax.axis_index("s") * 2 + lax.axis_index("c")
        bi = wid // wpb
        nbase = (wid % wpb) * ppw
        base = wid * ppw
        pltpu.sync_copy(s_hbm.at[bi], sv)
        pltpu.sync_copy(p_hbm.at[bi, :, pl.ds(nbase, ppw)], pv)
        pltpu.sync_copy(idx_hbm.at[wid], iv)

        row0 = jnp.zeros((16,), jnp.int32)
        row1 = jnp.full((16,), 1, jnp.int32)
        row2 = jnp.full((16,), 2, jnp.int32)

        @pl.loop(0, ppw, step=16)
        def _chunk(c):
            sl = pl.ds(c, 16)
            pxv, pyv, pzv = pv[0, sl], pv[1, sl], pv[2, sl]
            i0 = iv[0, sl]
            c0x = plsc.load_gather(sv, [row0, i0])
            c0y = plsc.load_gather(sv, [row1, i0])
            c0z = plsc.load_gather(sv, [row2, i0])
            d0x, d0y, d0z = pxv - c0x, pyv - c0y, pzv - c0z
            d0 = d0x * d0x + d0y * d0y + d0z * d0z
            acc = jnp.full((16,), jnp.inf, jnp.float32)
            for j in range(1, _K):
                ij = iv[j, sl]
                cjx = plsc.load_gather(sv, [row0, ij])
                cjy = plsc.load_gather(sv, [row1, ij])
                cjz = plsc.load_gather(sv, [row2, ij])
                ex, ey, ez = cjx - c0x, cjy - c0y, cjz - c0z
                lsq = ex * ex + ey * ey + ez * ez
                qx, qy, qz = pxv - cjx, pyv - cjy, pzv - cjz
                dj = qx * qx + qy * qy + qz * qz
                diff = dj - d0
                val = (diff * diff) / (4.0 * lsq)
                acc = jnp.minimum(acc, val)
            ov[sl] = acc

        pltpu.sync_copy(ov, out_hbm.at[pl.ds(base, ppw)])

    return body(idx_w, points_t, spoints_t)


def kernel(points, spoints):
    b, n, _ = points.shape
    total = b * n
    ppw = total // _NW
    spoints_t = jnp.transpose(spoints, (0, 2, 1))
    idx = _topk_indices(points, spoints_t)
    idx_w = idx.reshape(_NW, ppw, _K).transpose(0, 2, 1)
    points_t = jnp.transpose(points, (0, 2, 1))
    out = _edge_min(idx_w, points_t, spoints_t)
    return out.reshape(b, n)


# speedup vs baseline: 1.7688x; 1.0035x over previous
import dataclasses
import functools

import jax
import jax.numpy as jnp
from jax import lax
from jax.experimental import pallas as pl
from jax.experimental.pallas import tpu as pltpu
from jax.experimental.pallas import tpu_sc as plsc

_K = 11
_BN = 512
_NW = 32


def _topk_body(p_ref, st_ref, idx_ref):
    p = p_ref[0]
    st = st_ref[0]
    m = st.shape[1]
    bn = p.shape[0]
    nc = m // 128
    dist = None
    for d in range(3):
        diff = p[:, d:d + 1] - st[d:d + 1, :]
        dist = diff * diff if dist is None else dist + diff * diff
    bits = lax.bitcast_convert_type(dist, jnp.int32)
    chunks = [
        lax.bitcast_convert_type(
            (bits[:, c * 128:(c + 1) * 128] & jnp.int32(~15)) | jnp.int32(c),
            jnp.float32)
        for c in range(nc)
    ]
    lane_iota = lax.convert_element_type(
        lax.broadcasted_iota(jnp.int32, (bn, 128), 1), jnp.float32)
    inf = jnp.float32(jnp.inf)
    prev = None
    for k in range(_K):
        acc = None
        for c in range(nc):
            cand = (chunks[c] if prev is None
                    else jnp.where(chunks[c] > prev, chunks[c], inf))
            acc = cand if acc is None else jnp.minimum(acc, cand)
        mn = jnp.min(acc, axis=1, keepdims=True)
        lane = lax.convert_element_type(
            jnp.min(jnp.where(acc == mn, lane_iota, jnp.float32(128.0)),
                    axis=1, keepdims=True), jnp.int32)
        mn_bits = lax.bitcast_convert_type(mn, jnp.int32)
        idx_ref[0, :, k:k + 1] = (mn_bits & 15) * 128 + lane
        prev = mn


def _topk_indices(points, spoints_t, interpret=False):
    b, n, _ = points.shape
    m = spoints_t.shape[2]
    return pl.pallas_call(
        _topk_body,
        grid=(b, n // _BN),
        in_specs=[
            pl.BlockSpec((1, _BN, 3), lambda bi, i: (bi, i, 0)),
            pl.BlockSpec((1, 3, m), lambda bi, i: (bi, 0, 0)),
        ],
        out_specs=pl.BlockSpec((1, _BN, _K), lambda bi, i: (bi, i, 0)),
        out_shape=jax.ShapeDtypeStruct((b, n, _K), jnp.int32),
        interpret=interpret,
    )(points, spoints_t)


def _edge_min(idx_w, points_t, spoints_t):
    b, _, n = points_t.shape
    m = spoints_t.shape[2]
    total = b * n
    ppw = total // _NW
    wpb = _NW // b
    mesh = plsc.VectorSubcoreMesh(core_axis_name="c", subcore_axis_name="s")
    cp = pltpu.CompilerParams()
    if "needs_layout_passes" in pltpu.CompilerParams.__dataclass_fields__:
        cp = dataclasses.replace(cp, needs_layout_passes=False)

    @functools.partial(
        pl.kernel,
        out_type=jax.ShapeDtypeStruct((total,), jnp.float32),
        mesh=mesh,
        compiler_params=cp,
        scratch_types=[
            pltpu.VMEM((3, m), jnp.float32),
            pltpu.VMEM((3, ppw), jnp.float32),
            pltpu.VMEM((_K, ppw), jnp.int32),
            pltpu.VMEM((ppw,), jnp.float32),
        ],
    )
    def body(idx_hbm, p_hbm, s_hbm, out_hbm, sv, pv, iv, ov):
        wid = lax.axis_index("s") * 2 + lax.axis_index("c")
        bi = wid // wpb
        nbase = (wid % wpb) * ppw
        base = wid * ppw
        pltpu.sync_copy(s_hbm.at[bi], sv)
        pltpu.sync_copy(p_hbm.at[bi, :, pl.ds(nbase, ppw)], pv)
        pltpu.sync_copy(idx_hbm.at[wid], iv)

        row0 = jnp.zeros((16,), jnp.int32)
        row1 = jnp.full((16,), 1, jnp.int32)
        row2 = jnp.full((16,), 2, jnp.int32)

        @pl.loop(0, ppw, step=16)
        def _chunk(c):
            sl = pl.ds(c, 16)
            pxv, pyv, pzv = pv[0, sl], pv[1, sl], pv[2, sl]
            i0 = iv[0, sl]
            c0x = plsc.load_gather(sv, [row0, i0])
            c0y = plsc.load_gather(sv, [row1, i0])
            c0z = plsc.load_gather(sv, [row2, i0])
            d0x, d0y, d0z = pxv - c0x, pyv - c0y, pzv - c0z
            d0 = d0x * d0x + d0y * d0y + d0z * d0z
            acc = jnp.full((16,), jnp.inf, jnp.float32)
            for j in range(1, _K):
                ij = iv[j, sl]
                cjx = plsc.load_gather(sv, [row0, ij])
                cjy = plsc.load_gather(sv, [row1, ij])
                cjz = plsc.load_gather(sv, [row2, ij])
                ex, ey, ez = cjx - c0x, cjy - c0y, cjz - c0z
                lsq = ex * ex + ey * ey + ez * ez
                qx, qy, qz = pxv - cjx, pyv - cjy, pzv - cjz
                dj = qx * qx + qy * qy + qz * qz
                diff = dj - d0
                val = (diff * diff) / (4.0 * lsq)
                acc = jnp.minimum(acc, val)
            ov[sl] = acc

        pltpu.sync_copy(ov, out_hbm.at[pl.ds(base, ppw)])

    return body(idx_w, points_t, spoints_t)


def kernel(points, spoints):
    b, n, _ = points.shape
    total = b * n
    ppw = total // _NW
    spoints_t = jnp.transpose(spoints, (0, 2, 1))
    idx = _topk_indices(points, spoints_t)
    idx_w = idx.reshape(_NW, ppw, _K).transpose(0, 2, 1)
    points_t = jnp.transpose(points, (0, 2, 1))
    out = _edge_min(idx_w, points_t, spoints_t)
    return out.reshape(b, n)


# speedup vs baseline: 2.1183x; 1.1976x over previous
import dataclasses
import functools

import jax
import jax.numpy as jnp
from jax import lax
from jax.experimental import pallas as pl
from jax.experimental.pallas import tpu as pltpu
from jax.experimental.pallas import tpu_sc as plsc

_K = 11
_BN = 512
_NW = 32


def _topk_body(p_ref, st_ref, idx_ref):
    p = p_ref[0]
    st = st_ref[0]
    m = st.shape[1]
    bn = p.shape[0]
    nc = m // 128
    dist = None
    for d in range(3):
        diff = p[:, d:d + 1] - st[d:d + 1, :]
        dist = diff * diff if dist is None else dist + diff * diff
    bits = lax.bitcast_convert_type(dist, jnp.int32)
    chunks = [
        lax.bitcast_convert_type(
            (bits[:, c * 128:(c + 1) * 128] & jnp.int32(~15)) | jnp.int32(c),
            jnp.float32)
        for c in range(nc)
    ]
    lane_iota = lax.convert_element_type(
        lax.broadcasted_iota(jnp.int32, (bn, 128), 1), jnp.float32)
    inf = jnp.float32(jnp.inf)
    depth = 6
    lists = [jnp.full((bn, 128), inf, jnp.float32) for _ in range(depth)]
    for c in range(nc):
        x = chunks[c]
        for i in range(depth):
            lo = jnp.minimum(lists[i], x)
            x = jnp.maximum(lists[i], x)
            lists[i] = lo
    for k in range(_K):
        head = lists[0]
        mn = jnp.min(head, axis=1, keepdims=True)
        lane_f = jnp.min(jnp.where(head == mn, lane_iota, jnp.float32(128.0)),
                         axis=1, keepdims=True)
        lane = lax.convert_element_type(lane_f, jnp.int32)
        mn_bits = lax.bitcast_convert_type(mn, jnp.int32)
        idx_ref[0, :, k:k + 1] = (mn_bits & 15) * 128 + lane
        if k < _K - 1:
            is_l = lane_iota == lane_f
            for i in range(depth - 1):
                lists[i] = jnp.where(is_l, lists[i + 1], lists[i])
            lists[depth - 1] = jnp.where(is_l, inf, lists[depth - 1])


def _topk_indices(points, spoints_t, interpret=False):
    b, n, _ = points.shape
    m = spoints_t.shape[2]
    return pl.pallas_call(
        _topk_body,
        grid=(b, n // _BN),
        in_specs=[
            pl.BlockSpec((1, _BN, 3), lambda bi, i: (bi, i, 0)),
            pl.BlockSpec((1, 3, m), lambda bi, i: (bi, 0, 0)),
        ],
        out_specs=pl.BlockSpec((1, _BN, _K), lambda bi, i: (bi, i, 0)),
        out_shape=jax.ShapeDtypeStruct((b, n, _K), jnp.int32),
        interpret=interpret,
    )(points, spoints_t)


def _edge_min(idx_w, points_t, spoints_t):
    b, _, n = points_t.shape
    m = spoints_t.shape[2]
    total = b * n
    ppw = total // _NW
    wpb = _NW // b
    mesh = plsc.VectorSubcoreMesh(core_axis_name="c", subcore_axis_name="s")
    cp = pltpu.CompilerParams()
    if "needs_layout_passes" in pltpu.CompilerParams.__dataclass_fields__:
        cp = dataclasses.replace(cp, needs_layout_passes=False)

    @functools.partial(
        pl.kernel,
        out_type=jax.ShapeDtypeStruct((total,), jnp.float32),
        mesh=mesh,
        compiler_params=cp,
        scratch_types=[
            pltpu.VMEM((3, m), jnp.float32),
            pltpu.VMEM((3, ppw), jnp.float32),
            pltpu.VMEM((_K, ppw), jnp.int32),
            pltpu.VMEM((ppw,), jnp.float32),
        ],
    )
    def body(idx_hbm, p_hbm, s_hbm, out_hbm, sv, pv, iv, ov):
        wid = lax.axis_index("s") * 2 + lax.axis_index("c")
        bi = wid // wpb
        nbase = (wid % wpb) * ppw
        base = wid * ppw
        pltpu.sync_copy(s_hbm.at[bi], sv)
        pltpu.sync_copy(p_hbm.at[bi, :, pl.ds(nbase, ppw)], pv)
        pltpu.sync_copy(idx_hbm.at[wid], iv)

        row0 = jnp.zeros((16,), jnp.int32)
        row1 = jnp.full((16,), 1, jnp.int32)
        row2 = jnp.full((16,), 2, jnp.int32)

        @pl.loop(0, ppw, step=16)
        def _chunk(c):
            sl = pl.ds(c, 16)
            pxv, pyv, pzv = pv[0, sl], pv[1, sl], pv[2, sl]
            i0 = iv[0, sl]
            c0x = plsc.load_gather(sv, [row0, i0])
            c0y = plsc.load_gather(sv, [row1, i0])
            c0z = plsc.load_gather(sv, [row2, i0])
            d0x, d0y, d0z = pxv - c0x, pyv - c0y, pzv - c0z
            d0 = d0x * d0x + d0y * d0y + d0z * d0z
            acc = jnp.full((16,), jnp.inf, jnp.float32)
            for j in range(1, _K):
                ij = iv[j, sl]
                cjx = plsc.load_gather(sv, [row0, ij])
                cjy = plsc.load_gather(sv, [row1, ij])
                cjz = plsc.load_gather(sv, [row2, ij])
                ex, ey, ez = cjx - c0x, cjy - c0y, cjz - c0z
                lsq = ex * ex + ey * ey + ez * ez
                qx, qy, qz = pxv - cjx, pyv - cjy, pzv - cjz
                dj = qx * qx + qy * qy + qz * qz
                diff = dj - d0
                val = (diff * diff) / (4.0 * lsq)
                acc = jnp.minimum(acc, val)
            ov[sl] = acc

        pltpu.sync_copy(ov, out_hbm.at[pl.ds(base, ppw)])

    return body(idx_w, points_t, spoints_t)


def kernel(points, spoints):
    b, n, _ = points.shape
    total = b * n
    ppw = total // _NW
    spoints_t = jnp.transpose(spoints, (0, 2, 1))
    idx = _topk_indices(points, spoints_t)
    idx_w = idx.reshape(_NW, ppw, _K).transpose(0, 2, 1)
    points_t = jnp.transpose(points, (0, 2, 1))
    out = _edge_min(idx_w, points_t, spoints_t)
    return out.reshape(b, n)


# speedup vs baseline: 2.1313x; 1.0062x over previous
import dataclasses
import functools

import jax
import jax.numpy as jnp
from jax import lax
from jax.experimental import pallas as pl
from jax.experimental.pallas import tpu as pltpu
from jax.experimental.pallas import tpu_sc as plsc

_K = 11
_BN = 1024
_NW = 32


def _topk_body(p_ref, st_ref, idx_ref):
    p = p_ref[0]
    st = st_ref[0]
    m = st.shape[1]
    bn = p.shape[0]
    nc = m // 128
    dist = None
    for d in range(3):
        diff = p[:, d:d + 1] - st[d:d + 1, :]
        dist = diff * diff if dist is None else dist + diff * diff
    bits = lax.bitcast_convert_type(dist, jnp.int32)
    chunks = [
        lax.bitcast_convert_type(
            (bits[:, c * 128:(c + 1) * 128] & jnp.int32(~15)) | jnp.int32(c),
            jnp.float32)
        for c in range(nc)
    ]
    lane_iota = lax.convert_element_type(
        lax.broadcasted_iota(jnp.int32, (bn, 128), 1), jnp.float32)
    inf = jnp.float32(jnp.inf)
    depth = 6
    lists = [jnp.full((bn, 128), inf, jnp.float32) for _ in range(depth)]
    for c in range(nc):
        x = chunks[c]
        for i in range(depth):
            lo = jnp.minimum(lists[i], x)
            x = jnp.maximum(lists[i], x)
            lists[i] = lo
    for k in range(_K):
        head = lists[0]
        mn = jnp.min(head, axis=1, keepdims=True)
        lane_f = jnp.min(jnp.where(head == mn, lane_iota, jnp.float32(128.0)),
                         axis=1, keepdims=True)
        lane = lax.convert_element_type(lane_f, jnp.int32)
        mn_bits = lax.bitcast_convert_type(mn, jnp.int32)
        idx_ref[0, :, k:k + 1] = (mn_bits & 15) * 128 + lane
        if k < _K - 1:
            is_l = lane_iota == lane_f
            for i in range(depth - 1):
                lists[i] = jnp.where(is_l, lists[i + 1], lists[i])
            lists[depth - 1] = jnp.where(is_l, inf, lists[depth - 1])


def _topk_indices(points, spoints_t, interpret=False):
    b, n, _ = points.shape
    m = spoints_t.shape[2]
    return pl.pallas_call(
        _topk_body,
        grid=(b, n // _BN),
        in_specs=[
            pl.BlockSpec((1, _BN, 3), lambda bi, i: (bi, i, 0)),
            pl.BlockSpec((1, 3, m), lambda bi, i: (bi, 0, 0)),
        ],
        out_specs=pl.BlockSpec((1, _BN, _K), lambda bi, i: (bi, i, 0)),
        out_shape=jax.ShapeDtypeStruct((b, n, _K), jnp.int32),
        interpret=interpret,
    )(points, spoints_t)


def _edge_min(idx_w, points_t, spoints_t):
    b, _, n = points_t.shape
    m = spoints_t.shape[2]
    total = b * n
    ppw = total // _NW
    wpb = _NW // b
    mesh = plsc.VectorSubcoreMesh(core_axis_name="c", subcore_axis_name="s")
    cp = pltpu.CompilerParams()
    if "needs_layout_passes" in pltpu.CompilerParams.__dataclass_fields__:
        cp = dataclasses.replace(cp, needs_layout_passes=False)

    @functools.partial(
        pl.kernel,
        out_type=jax.ShapeDtypeStruct((total,), jnp.float32),
        mesh=mesh,
        compiler_params=cp,
        scratch_types=[
            pltpu.VMEM((3, m), jnp.float32),
            pltpu.VMEM((3, ppw), jnp.float32),
            pltpu.VMEM((_K, ppw), jnp.int32),
            pltpu.VMEM((ppw,), jnp.float32),
        ],
    )
    def body(idx_hbm, p_hbm, s_hbm, out_hbm, sv, pv, iv, ov):
        wid = lax.axis_index("s") * 2 + lax.axis_index("c")
        bi = wid // wpb
        nbase = (wid % wpb) * ppw
        base = wid * ppw
        pltpu.sync_copy(s_hbm.at[bi], sv)
        pltpu.sync_copy(p_hbm.at[bi, :, pl.ds(nbase, ppw)], pv)
        pltpu.sync_copy(idx_hbm.at[wid], iv)

        row0 = jnp.zeros((16,), jnp.int32)
        row1 = jnp.full((16,), 1, jnp.int32)
        row2 = jnp.full((16,), 2, jnp.int32)

        @pl.loop(0, ppw, step=16)
        def _chunk(c):
            sl = pl.ds(c, 16)
            pxv, pyv, pzv = pv[0, sl], pv[1, sl], pv[2, sl]
            i0 = iv[0, sl]
            c0x = plsc.load_gather(sv, [row0, i0])
            c0y = plsc.load_gather(sv, [row1, i0])
            c0z = plsc.load_gather(sv, [row2, i0])
            d0x, d0y, d0z = pxv - c0x, pyv - c0y, pzv - c0z
            d0 = d0x * d0x + d0y * d0y + d0z * d0z
            acc = jnp.full((16,), jnp.inf, jnp.float32)
            for j in range(1, _K):
                ij = iv[j, sl]
                cjx = plsc.load_gather(sv, [row0, ij])
                cjy = plsc.load_gather(sv, [row1, ij])
                cjz = plsc.load_gather(sv, [row2, ij])
                ex, ey, ez = cjx - c0x, cjy - c0y, cjz - c0z
                lsq = ex * ex + ey * ey + ez * ez
                qx, qy, qz = pxv - cjx, pyv - cjy, pzv - cjz
                dj = qx * qx + qy * qy + qz * qz
                diff = dj - d0
                val = (diff * diff) / (4.0 * lsq)
                acc = jnp.minimum(acc, val)
            ov[sl] = acc

        pltpu.sync_copy(ov, out_hbm.at[pl.ds(base, ppw)])

    return body(idx_w, points_t, spoints_t)


def kernel(points, spoints):
    b, n, _ = points.shape
    total = b * n
    ppw = total // _NW
    spoints_t = jnp.transpose(spoints, (0, 2, 1))
    idx = _topk_indices(points, spoints_t)
    idx_w = idx.reshape(_NW, ppw, _K).transpose(0, 2, 1)
    points_t = jnp.transpose(points, (0, 2, 1))
    out = _edge_min(idx_w, points_t, spoints_t)
    return out.reshape(b, n)
